# baseline (speedup 1.0000x reference)
import jax
import jax.numpy as jnp
from jax.experimental import pallas as pl

MAX_POS_ = 2048
HIDDEN_ = 768
BATCH_ = 4

BP = 512


def _add_body(hid_ref, pos_ref, out_ref):
    out_ref[...] = hid_ref[...] + pos_ref[...]


def kernel(hidden_states, pos_table):
    grid = (MAX_POS_ // BP,)
    return pl.pallas_call(
        _add_body,
        grid=grid,
        in_specs=[
            pl.BlockSpec((BATCH_, BP, HIDDEN_), lambda i: (0, i, 0)),
            pl.BlockSpec((BP, HIDDEN_), lambda i: (i, 0)),
        ],
        out_specs=pl.BlockSpec((BATCH_, BP, HIDDEN_), lambda i: (0, i, 0)),
        out_shape=jax.ShapeDtypeStruct((BATCH_, MAX_POS_, HIDDEN_), jnp.float32),
    )(hidden_states, pos_table)
